# baseline (speedup 1.0000x reference)
import jax
import jax.numpy as jnp
from jax.experimental import pallas as pl

B, N, E, H, K = 1024, 200, 128, 256, 20
BBLK = 64
HEAD_MXU = True


def _body(neib_ref, sg_ref, pad_ref, nei_num_ref, sub_num_ref,
          w2mat_ref, b2_ref, wn_ref, bn_ref,
          id_ref, prob_ref):
    M = BBLK * N
    bf16 = jnp.bfloat16
    w1 = wn_ref[:, :H]
    w2 = wn_ref[:, H:]

    nb = neib_ref[...].reshape(M, E).astype(bf16)
    w2mat = w2mat_ref[...].astype(bf16)
    nei = jax.lax.dot_general(nb, w2mat, (((1,), (1,)), ((), ())),
                              preferred_element_type=jnp.float32)
    nei = nei + b2_ref[...]

    if HEAD_MXU:
        w1b = jnp.broadcast_to(w1.astype(bf16), (8, H))
        t1 = jax.lax.dot_general(nei.astype(bf16), w1b,
                                 (((1,), (1,)), ((), ())),
                                 preferred_element_type=jnp.float32)
        t1 = t1[:, 0:1].reshape(BBLK, N)
    else:
        nei3 = nei.reshape(BBLK, N, H).astype(bf16).astype(jnp.float32)
        w1f = w1.astype(bf16).astype(jnp.float32)
        t1 = jnp.sum(nei3 * w1f.reshape(1, 1, H), axis=2)
    w2b = jnp.broadcast_to(w2.astype(bf16), (8, H))
    t2 = jax.lax.dot_general(sg_ref[...].astype(bf16), w2b,
                             (((1,), (1,)), ((), ())),
                             preferred_element_type=jnp.float32)[:, 0:1]

    score = jax.nn.sigmoid(t1 + t2 + bn_ref[0, 0]) + pad_ref[...]

    m = jnp.max(score, axis=1, keepdims=True)
    ex = jnp.exp(score - m)
    z = jnp.sum(ex, axis=1, keepdims=True)
    att = ex / z

    iota = jax.lax.broadcasted_iota(jnp.int32, (BBLK, N), 1)
    work = att
    selmask = jnp.zeros((BBLK, N), dtype=jnp.bool_)
    topsum = jnp.zeros((BBLK, 1), dtype=jnp.float32)
    for _ in range(K):
        mx = jnp.max(work, axis=1, keepdims=True)
        eq = work == mx
        idxk = jnp.min(jnp.where(eq, iota, N), axis=1, keepdims=True)
        chosen = iota == idxk
        selmask = selmask | chosen
        topsum = topsum + mx
        work = jnp.where(chosen, -1.0, work)

    self32 = selmask.astype(jnp.float32)
    lower_tri = (jax.lax.broadcasted_iota(jnp.int32, (N, N), 0)
                 <= jax.lax.broadcasted_iota(jnp.int32, (N, N), 1)).astype(jnp.float32)
    cs = jax.lax.dot_general(self32, lower_tri,
                             (((1,), (0,)), ((), ())),
                             preferred_element_type=jnp.float32)
    csi = cs.astype(jnp.int32)
    cols = []
    for k in range(K):
        sel_k = selmask & (csi == (k + 1))
        idx_k = jnp.sum(jnp.where(sel_k, iota, 0), axis=1, keepdims=True)
        cols.append(idx_k)
    idx_sorted = jnp.concatenate(cols, axis=1)

    id_ref[...] = idx_sorted + sub_num_ref[...]
    nei_num_f = nei_num_ref[...].astype(jnp.float32)
    prob_ref[...] = topsum * nei_num_f * (1.0 / K)


@jax.jit
def kernel(x, sub_graph, neibour, mask, nei_pad_mask, nei_num, sub_num, W2, b2, Wn, bn):
    del x, mask
    grid = (B // BBLK,)
    out_id, out_prob = pl.pallas_call(
        _body,
        grid=grid,
        in_specs=[
            pl.BlockSpec((BBLK, N, E), lambda i: (i, 0, 0)),
            pl.BlockSpec((BBLK, H), lambda i: (i, 0)),
            pl.BlockSpec((BBLK, N), lambda i: (i, 0)),
            pl.BlockSpec((BBLK, 1), lambda i: (i, 0)),
            pl.BlockSpec((BBLK, 1), lambda i: (i, 0)),
            pl.BlockSpec((H, E), lambda i: (0, 0)),
            pl.BlockSpec((1, H), lambda i: (0, 0)),
            pl.BlockSpec((1, 2 * H), lambda i: (0, 0)),
            pl.BlockSpec((1, 1), lambda i: (0, 0)),
        ],
        out_specs=[
            pl.BlockSpec((BBLK, K), lambda i: (i, 0)),
            pl.BlockSpec((BBLK, 1), lambda i: (i, 0)),
        ],
        out_shape=[
            jax.ShapeDtypeStruct((B, K), jnp.int32),
            jax.ShapeDtypeStruct((B, 1), jnp.float32),
        ],
    )(neibour, sub_graph, nei_pad_mask, nei_num, sub_num,
      W2, b2.reshape(1, H), Wn, bn.reshape(1, 1))
    return out_id, out_prob.reshape(B)


# speedup vs baseline: 1.3009x; 1.3009x over previous
import jax
import jax.numpy as jnp
from jax.experimental import pallas as pl
from jax.experimental.pallas import tpu as pltpu

B, N, E, H, K = 1024, 200, 128, 256, 20
BBLK = 64
NSTEPS = B // BBLK


def _score_phase(neib_ref, sg_ref, pad_ref, wn_ref, w2mat_ref, b2_ref, bn_ref,
                 score_scr, parity):
    M = BBLK * N
    bf16 = jnp.bfloat16
    w1 = wn_ref[:, :H]
    w2 = wn_ref[:, H:]

    nb = neib_ref[...].reshape(M, E).astype(bf16)
    w2mat = w2mat_ref[...].astype(bf16)
    nei = jax.lax.dot_general(nb, w2mat, (((1,), (1,)), ((), ())),
                              preferred_element_type=jnp.float32)
    nei = nei + b2_ref[...]

    w1b = jnp.broadcast_to(w1.astype(bf16), (8, H))
    t1 = jax.lax.dot_general(nei.astype(bf16), w1b,
                             (((1,), (1,)), ((), ())),
                             preferred_element_type=jnp.float32)
    t1 = t1[:, 0:1].reshape(BBLK, N)
    w2b = jnp.broadcast_to(w2.astype(bf16), (8, H))
    t2 = jax.lax.dot_general(sg_ref[...].astype(bf16), w2b,
                             (((1,), (1,)), ((), ())),
                             preferred_element_type=jnp.float32)[:, 0:1]

    score = jax.nn.sigmoid(t1 + t2 + bn_ref[0, 0]) + pad_ref[...]
    score_scr[pl.ds(parity, 1)] = score.reshape(1, BBLK, N)


def _select_phase(nei_num_ref, sub_num_ref, id_ref, prob_ref, score_scr, parity):
    score = score_scr[pl.ds(parity, 1)].reshape(BBLK, N)

    m = jnp.max(score, axis=1, keepdims=True)
    ex = jnp.exp(score - m)
    z = jnp.sum(ex, axis=1, keepdims=True)
    att = ex / z

    iota = jax.lax.broadcasted_iota(jnp.int32, (BBLK, N), 1)
    work = att
    selmask = jnp.zeros((BBLK, N), dtype=jnp.bool_)
    topsum = jnp.zeros((BBLK, 1), dtype=jnp.float32)
    for _ in range(K):
        mx = jnp.max(work, axis=1, keepdims=True)
        eq = work == mx
        idxk = jnp.min(jnp.where(eq, iota, N), axis=1, keepdims=True)
        chosen = iota == idxk
        selmask = selmask | chosen
        topsum = topsum + mx
        work = jnp.where(chosen, -1.0, work)

    self32 = selmask.astype(jnp.float32)
    lower_tri = (jax.lax.broadcasted_iota(jnp.int32, (N, N), 0)
                 <= jax.lax.broadcasted_iota(jnp.int32, (N, N), 1)).astype(jnp.float32)
    cs = jax.lax.dot_general(self32, lower_tri,
                             (((1,), (0,)), ((), ())),
                             preferred_element_type=jnp.float32)
    rank = (cs * self32).astype(jnp.int32)
    kio = jax.lax.broadcasted_iota(jnp.int32, (BBLK, N, K), 2)
    onehot = (rank[:, :, None] == kio + 1)
    iota3 = jax.lax.broadcasted_iota(jnp.int32, (BBLK, N, K), 1)
    idx_sorted = jnp.sum(jnp.where(onehot, iota3, 0), axis=1)

    id_ref[...] = idx_sorted + sub_num_ref[...]
    nei_num_f = nei_num_ref[...].astype(jnp.float32)
    prob_ref[...] = topsum * nei_num_f * (1.0 / K)


def _body(neib_ref, sg_ref, pad_ref, nei_num_ref, sub_num_ref,
          w2mat_ref, b2_ref, wn_ref, bn_ref,
          id_ref, prob_ref, score_scr):
    i = pl.program_id(0)
    parity = jax.lax.rem(i, 2)

    @pl.when(i < NSTEPS)
    def _():
        _score_phase(neib_ref, sg_ref, pad_ref, wn_ref, w2mat_ref, b2_ref,
                     bn_ref, score_scr, parity)

    @pl.when(i > 0)
    def _():
        _select_phase(nei_num_ref, sub_num_ref, id_ref, prob_ref, score_scr,
                      1 - parity)


def _clamp_hi(i):
    return jnp.minimum(i, NSTEPS - 1)


def _lag(i):
    return jnp.maximum(i - 1, 0)


@jax.jit
def kernel(x, sub_graph, neibour, mask, nei_pad_mask, nei_num, sub_num, W2, b2, Wn, bn):
    del x, mask
    grid = (NSTEPS + 1,)
    out_id, out_prob = pl.pallas_call(
        _body,
        grid=grid,
        in_specs=[
            pl.BlockSpec((BBLK, N, E), lambda i: (_clamp_hi(i), 0, 0)),
            pl.BlockSpec((BBLK, H), lambda i: (_clamp_hi(i), 0)),
            pl.BlockSpec((BBLK, N), lambda i: (_clamp_hi(i), 0)),
            pl.BlockSpec((BBLK, 1), lambda i: (_lag(i), 0)),
            pl.BlockSpec((BBLK, 1), lambda i: (_lag(i), 0)),
            pl.BlockSpec((H, E), lambda i: (0, 0)),
            pl.BlockSpec((1, H), lambda i: (0, 0)),
            pl.BlockSpec((1, 2 * H), lambda i: (0, 0)),
            pl.BlockSpec((1, 1), lambda i: (0, 0)),
        ],
        out_specs=[
            pl.BlockSpec((BBLK, K), lambda i: (_lag(i), 0)),
            pl.BlockSpec((BBLK, 1), lambda i: (_lag(i), 0)),
        ],
        out_shape=[
            jax.ShapeDtypeStruct((B, K), jnp.int32),
            jax.ShapeDtypeStruct((B, 1), jnp.float32),
        ],
        scratch_shapes=[pltpu.VMEM((2, BBLK, N), jnp.float32)],
    )(neibour, sub_graph, nei_pad_mask, nei_num, sub_num,
      W2, b2.reshape(1, H), Wn, bn.reshape(1, 1))
    return out_id, out_prob.reshape(B)


# speedup vs baseline: 1.8707x; 1.4380x over previous
import jax
import jax.numpy as jnp
from jax.experimental import pallas as pl
from jax.experimental.pallas import tpu as pltpu

B, N, E, H, K = 1024, 200, 128, 256, 20
BBLK = 64
NSTEPS = B // BBLK


def _score_phase(neib_ref, sg_ref, pad_ref, wn_ref, w2mat_ref, b2_ref, bn_ref,
                 score_scr, parity):
    M = BBLK * N
    bf16 = jnp.bfloat16
    w1 = wn_ref[:, :H]
    w2 = wn_ref[:, H:]

    nb = neib_ref[...].reshape(M, E).astype(bf16)
    w2mat = w2mat_ref[...].astype(bf16)
    nei = jax.lax.dot_general(nb, w2mat, (((1,), (1,)), ((), ())),
                              preferred_element_type=jnp.float32)
    nei = nei + b2_ref[...]

    w1b = jnp.broadcast_to(w1.astype(bf16), (8, H))
    t1 = jax.lax.dot_general(nei.astype(bf16), w1b,
                             (((1,), (1,)), ((), ())),
                             preferred_element_type=jnp.float32)
    t1 = t1[:, 0:1].reshape(BBLK, N)
    w2b = jnp.broadcast_to(w2.astype(bf16), (8, H))
    t2 = jax.lax.dot_general(sg_ref[...].astype(bf16), w2b,
                             (((1,), (1,)), ((), ())),
                             preferred_element_type=jnp.float32)[:, 0:1]

    score = jax.nn.sigmoid(t1 + t2 + bn_ref[0, 0]) + pad_ref[...]
    score_scr[pl.ds(parity, 1)] = score.reshape(1, BBLK, N)


def _select_phase(nei_num_ref, sub_num_ref, id_ref, prob_ref, score_scr, parity):
    score = score_scr[pl.ds(parity, 1)].reshape(BBLK, N)

    m = jnp.max(score, axis=1, keepdims=True)
    ex = jnp.exp(score - m)
    z = jnp.sum(ex, axis=1, keepdims=True)
    att = ex / z

    iota = jax.lax.broadcasted_iota(jnp.int32, (BBLK, N), 1)
    work = att
    topsum = jnp.zeros((BBLK, 1), dtype=jnp.float32)
    cols = []
    for _ in range(K):
        mx = jnp.max(work, axis=1, keepdims=True)
        eq = work == mx
        idxk = jnp.min(jnp.where(eq, iota, N), axis=1, keepdims=True)
        cols.append(idxk)
        topsum = topsum + mx
        work = jnp.where(iota == idxk, -1.0, work)
    idxmat = jnp.concatenate(cols, axis=1)

    ii = idxmat[:, :, None]
    jj = idxmat[:, None, :]
    rank = jnp.sum((jj < ii).astype(jnp.int32), axis=2)
    ko = jax.lax.broadcasted_iota(jnp.int32, (BBLK, K, K), 2)
    onehot = rank[:, :, None] == ko
    idx_sorted = jnp.sum(jnp.where(onehot, ii, 0), axis=1)

    id_ref[...] = idx_sorted + sub_num_ref[...]
    nei_num_f = nei_num_ref[...].astype(jnp.float32)
    prob_ref[...] = topsum * nei_num_f * (1.0 / K)


def _body(neib_ref, sg_ref, pad_ref, nei_num_ref, sub_num_ref,
          w2mat_ref, b2_ref, wn_ref, bn_ref,
          id_ref, prob_ref, score_scr):
    i = pl.program_id(0)
    parity = jax.lax.rem(i, 2)

    _select_phase(nei_num_ref, sub_num_ref, id_ref, prob_ref, score_scr,
                  1 - parity)
    _score_phase(neib_ref, sg_ref, pad_ref, wn_ref, w2mat_ref, b2_ref,
                 bn_ref, score_scr, parity)


def _clamp_hi(i):
    return jnp.minimum(i, NSTEPS - 1)


def _lag(i):
    return jnp.maximum(i - 1, 0)


@jax.jit
def kernel(x, sub_graph, neibour, mask, nei_pad_mask, nei_num, sub_num, W2, b2, Wn, bn):
    del x, mask
    grid = (NSTEPS + 1,)
    out_id, out_prob = pl.pallas_call(
        _body,
        grid=grid,
        in_specs=[
            pl.BlockSpec((BBLK, N, E), lambda i: (_clamp_hi(i), 0, 0)),
            pl.BlockSpec((BBLK, H), lambda i: (_clamp_hi(i), 0)),
            pl.BlockSpec((BBLK, N), lambda i: (_clamp_hi(i), 0)),
            pl.BlockSpec((BBLK, 1), lambda i: (_lag(i), 0)),
            pl.BlockSpec((BBLK, 1), lambda i: (_lag(i), 0)),
            pl.BlockSpec((H, E), lambda i: (0, 0)),
            pl.BlockSpec((1, H), lambda i: (0, 0)),
            pl.BlockSpec((1, 2 * H), lambda i: (0, 0)),
            pl.BlockSpec((1, 1), lambda i: (0, 0)),
        ],
        out_specs=[
            pl.BlockSpec((BBLK, K), lambda i: (_lag(i), 0)),
            pl.BlockSpec((BBLK, 1), lambda i: (_lag(i), 0)),
        ],
        out_shape=[
            jax.ShapeDtypeStruct((B, K), jnp.int32),
            jax.ShapeDtypeStruct((B, 1), jnp.float32),
        ],
        scratch_shapes=[pltpu.VMEM((2, BBLK, N), jnp.float32)],
    )(neibour, sub_graph, nei_pad_mask, nei_num, sub_num,
      W2, b2.reshape(1, H), Wn, bn.reshape(1, 1))
    return out_id, out_prob.reshape(B)


# speedup vs baseline: 2.3001x; 1.2296x over previous
import jax
import jax.numpy as jnp
from jax.experimental import pallas as pl
from jax.experimental.pallas import tpu as pltpu

B, N, E, H, K = 1024, 200, 128, 256, 20
BBLK = 64
NSTEPS = B // BBLK


def _score_phase(neib_ref, sg_ref, pad_ref, wn_ref, w2mat_ref, b2_ref, bn_ref,
                 score_scr, parity):
    M = BBLK * N
    bf16 = jnp.bfloat16
    w1 = wn_ref[:, :H]
    w2 = wn_ref[:, H:]

    nb = neib_ref[...].reshape(M, E).astype(bf16)
    w2mat = w2mat_ref[...].astype(bf16)
    neiT = jax.lax.dot_general(w2mat, nb, (((1,), (1,)), ((), ())),
                               preferred_element_type=jnp.float32)
    neiT = neiT + b2_ref[...].reshape(H, 1)

    w1b = jnp.broadcast_to(w1.astype(bf16), (8, H))
    t1 = jax.lax.dot_general(w1b, neiT.astype(bf16),
                             (((1,), (0,)), ((), ())),
                             preferred_element_type=jnp.float32)
    t1row = t1[0:1, :]
    t1 = jnp.concatenate([t1row[:, b * N:(b + 1) * N] for b in range(BBLK)],
                         axis=0)
    w2b = jnp.broadcast_to(w2.astype(bf16), (8, H))
    t2 = jax.lax.dot_general(sg_ref[...].astype(bf16), w2b,
                             (((1,), (1,)), ((), ())),
                             preferred_element_type=jnp.float32)[:, 0:1]

    score = jax.nn.sigmoid(t1 + t2 + bn_ref[0, 0]) + pad_ref[...]
    score_scr[pl.ds(parity, 1)] = score.reshape(1, BBLK, N)


def _select_phase(nei_num_ref, sub_num_ref, id_ref, prob_ref, score_scr, parity):
    score = score_scr[pl.ds(parity, 1)].reshape(BBLK, N)

    m = jnp.max(score, axis=1, keepdims=True)
    ex = jnp.exp(score - m)
    z = jnp.sum(ex, axis=1, keepdims=True)
    att = ex / z

    iota = jax.lax.broadcasted_iota(jnp.int32, (BBLK, N), 1)
    work = att
    topsum = jnp.zeros((BBLK, 1), dtype=jnp.float32)
    cols = []
    for _ in range(K):
        mx = jnp.max(work, axis=1, keepdims=True)
        eq = work == mx
        idxk = jnp.min(jnp.where(eq, iota, N), axis=1, keepdims=True)
        cols.append(idxk)
        topsum = topsum + mx
        work = jnp.where(iota == idxk, -1.0, work)
    idxmat = jnp.concatenate(cols, axis=1)

    ii = idxmat[:, :, None]
    jj = idxmat[:, None, :]
    rank = jnp.sum((jj < ii).astype(jnp.int32), axis=2)
    ko = jax.lax.broadcasted_iota(jnp.int32, (BBLK, K, K), 2)
    onehot = rank[:, :, None] == ko
    idx_sorted = jnp.sum(jnp.where(onehot, ii, 0), axis=1)

    id_ref[...] = idx_sorted + sub_num_ref[...]
    nei_num_f = nei_num_ref[...].astype(jnp.float32)
    prob_ref[...] = topsum * nei_num_f * (1.0 / K)


def _body(neib_ref, sg_ref, pad_ref, nei_num_ref, sub_num_ref,
          w2mat_ref, b2_ref, wn_ref, bn_ref,
          id_ref, prob_ref, score_scr):
    i = pl.program_id(0)
    parity = jax.lax.rem(i, 2)

    _select_phase(nei_num_ref, sub_num_ref, id_ref, prob_ref, score_scr,
                  1 - parity)
    _score_phase(neib_ref, sg_ref, pad_ref, wn_ref, w2mat_ref, b2_ref,
                 bn_ref, score_scr, parity)


def _clamp_hi(i):
    return jnp.minimum(i, NSTEPS - 1)


def _lag(i):
    return jnp.maximum(i - 1, 0)


@jax.jit
def kernel(x, sub_graph, neibour, mask, nei_pad_mask, nei_num, sub_num, W2, b2, Wn, bn):
    del x, mask
    grid = (NSTEPS + 1,)
    out_id, out_prob = pl.pallas_call(
        _body,
        grid=grid,
        in_specs=[
            pl.BlockSpec((BBLK, N, E), lambda i: (_clamp_hi(i), 0, 0)),
            pl.BlockSpec((BBLK, H), lambda i: (_clamp_hi(i), 0)),
            pl.BlockSpec((BBLK, N), lambda i: (_clamp_hi(i), 0)),
            pl.BlockSpec((BBLK, 1), lambda i: (_lag(i), 0)),
            pl.BlockSpec((BBLK, 1), lambda i: (_lag(i), 0)),
            pl.BlockSpec((H, E), lambda i: (0, 0)),
            pl.BlockSpec((1, H), lambda i: (0, 0)),
            pl.BlockSpec((1, 2 * H), lambda i: (0, 0)),
            pl.BlockSpec((1, 1), lambda i: (0, 0)),
        ],
        out_specs=[
            pl.BlockSpec((BBLK, K), lambda i: (_lag(i), 0)),
            pl.BlockSpec((BBLK, 1), lambda i: (_lag(i), 0)),
        ],
        out_shape=[
            jax.ShapeDtypeStruct((B, K), jnp.int32),
            jax.ShapeDtypeStruct((B, 1), jnp.float32),
        ],
        scratch_shapes=[pltpu.VMEM((2, BBLK, N), jnp.float32)],
    )(neibour, sub_graph, nei_pad_mask, nei_num, sub_num,
      W2, b2.reshape(1, H), Wn, bn.reshape(1, 1))
    return out_id, out_prob.reshape(B)


# speedup vs baseline: 2.3432x; 1.0187x over previous
import jax
import jax.numpy as jnp
from jax.experimental import pallas as pl
from jax.experimental.pallas import tpu as pltpu

B, N, E, H, K = 1024, 200, 128, 256, 20
BBLK = 64
NSTEPS = B // BBLK


def _score_phase(neib_ref, sg_ref, pad_ref, wn_ref, w2mat_ref, b2_ref, bn_ref,
                 score_scr, parity):
    M = BBLK * N
    bf16 = jnp.bfloat16
    w1 = wn_ref[:, :H]
    w2 = wn_ref[:, H:]

    nb = neib_ref[...].reshape(M, E).astype(bf16)
    w2mat = w2mat_ref[...].astype(bf16)
    w1b = jnp.broadcast_to(w1.astype(bf16), (8, H))
    b2c = b2_ref[...].reshape(H, 1)

    NCHUNK = 4
    CM = M // NCHUNK
    rows_per_chunk = CM // N
    t1_parts = []
    for c in range(NCHUNK):
        nb_c = nb[c * CM:(c + 1) * CM, :]
        neiT = jax.lax.dot_general(w2mat, nb_c, (((1,), (1,)), ((), ())),
                                   preferred_element_type=jnp.float32)
        neiT = neiT + b2c
        t1c = jax.lax.dot_general(w1b, neiT.astype(bf16),
                                  (((1,), (0,)), ((), ())),
                                  preferred_element_type=jnp.float32)
        t1row = t1c[0:1, :]
        t1_parts.extend(t1row[:, b * N:(b + 1) * N]
                        for b in range(rows_per_chunk))
    t1 = jnp.concatenate(t1_parts, axis=0)
    w2b = jnp.broadcast_to(w2.astype(bf16), (8, H))
    t2 = jax.lax.dot_general(sg_ref[...].astype(bf16), w2b,
                             (((1,), (1,)), ((), ())),
                             preferred_element_type=jnp.float32)[:, 0:1]

    score = jax.nn.sigmoid(t1 + t2 + bn_ref[0, 0]) + pad_ref[...]
    score_scr[pl.ds(parity, 1)] = score.reshape(1, BBLK, N)


def _select_phase(nei_num_ref, sub_num_ref, id_ref, prob_ref, score_scr, parity):
    score_full = score_scr[pl.ds(parity, 1)].reshape(BBLK, N)
    idx_sorted, topsum = _select_half(score_full, BBLK)

    id_ref[...] = idx_sorted + sub_num_ref[...]
    nei_num_f = nei_num_ref[...].astype(jnp.float32)
    prob_ref[...] = topsum * nei_num_f * (1.0 / K)


def _select_half(score, rows):
    m = jnp.max(score, axis=1, keepdims=True)
    ex = jnp.exp(score - m)
    z = jnp.sum(ex, axis=1, keepdims=True)
    att = ex / z

    iota = jax.lax.broadcasted_iota(jnp.int32, (rows, N), 1)
    work = att
    topsum = jnp.zeros((rows, 1), dtype=jnp.float32)
    cols = []
    for _ in range(K):
        mx = jnp.max(work, axis=1, keepdims=True)
        eq = work == mx
        idxk = jnp.min(jnp.where(eq, iota, N), axis=1, keepdims=True)
        cols.append(idxk)
        topsum = topsum + mx
        work = jnp.where(iota == idxk, -1.0, work)
    idxmat = jnp.concatenate(cols, axis=1)

    ii = idxmat[:, :, None]
    jj = idxmat[:, None, :]
    rank = jnp.sum((jj < ii).astype(jnp.int32), axis=2)
    ko = jax.lax.broadcasted_iota(jnp.int32, (rows, K, K), 2)
    onehot = rank[:, :, None] == ko
    idx_sorted = jnp.sum(jnp.where(onehot, ii, 0), axis=1)
    return idx_sorted, topsum


def _body(neib_ref, sg_ref, pad_ref, nei_num_ref, sub_num_ref,
          w2mat_ref, b2_ref, wn_ref, bn_ref,
          id_ref, prob_ref, score_scr):
    i = pl.program_id(0)
    parity = jax.lax.rem(i, 2)

    _select_phase(nei_num_ref, sub_num_ref, id_ref, prob_ref, score_scr,
                  1 - parity)
    _score_phase(neib_ref, sg_ref, pad_ref, wn_ref, w2mat_ref, b2_ref,
                 bn_ref, score_scr, parity)


def _clamp_hi(i):
    return jnp.minimum(i, NSTEPS - 1)


def _lag(i):
    return jnp.maximum(i - 1, 0)


@jax.jit
def kernel(x, sub_graph, neibour, mask, nei_pad_mask, nei_num, sub_num, W2, b2, Wn, bn):
    del x, mask
    grid = (NSTEPS + 1,)
    out_id, out_prob = pl.pallas_call(
        _body,
        grid=grid,
        in_specs=[
            pl.BlockSpec((BBLK, N, E), lambda i: (_clamp_hi(i), 0, 0)),
            pl.BlockSpec((BBLK, H), lambda i: (_clamp_hi(i), 0)),
            pl.BlockSpec((BBLK, N), lambda i: (_clamp_hi(i), 0)),
            pl.BlockSpec((BBLK, 1), lambda i: (_lag(i), 0)),
            pl.BlockSpec((BBLK, 1), lambda i: (_lag(i), 0)),
            pl.BlockSpec((H, E), lambda i: (0, 0)),
            pl.BlockSpec((1, H), lambda i: (0, 0)),
            pl.BlockSpec((1, 2 * H), lambda i: (0, 0)),
            pl.BlockSpec((1, 1), lambda i: (0, 0)),
        ],
        out_specs=[
            pl.BlockSpec((BBLK, K), lambda i: (_lag(i), 0)),
            pl.BlockSpec((BBLK, 1), lambda i: (_lag(i), 0)),
        ],
        out_shape=[
            jax.ShapeDtypeStruct((B, K), jnp.int32),
            jax.ShapeDtypeStruct((B, 1), jnp.float32),
        ],
        scratch_shapes=[pltpu.VMEM((2, BBLK, N), jnp.float32)],
    )(neibour, sub_graph, nei_pad_mask, nei_num, sub_num,
      W2, b2.reshape(1, H), Wn, bn.reshape(1, 1))
    return out_id, out_prob.reshape(B)


# speedup vs baseline: 3.5583x; 1.5186x over previous
import jax
import jax.numpy as jnp
from jax.experimental import pallas as pl
from jax.experimental.pallas import tpu as pltpu

B, N, E, H, K = 1024, 200, 128, 256, 20
BBLK = 128
NSTEPS = B // BBLK


def _score_phase(neib_ref, sg_ref, pad_ref, wn_ref, w2mat_ref, b2_ref, bn_ref,
                 score_scr, parity):
    M = BBLK * N
    bf16 = jnp.bfloat16
    w1 = wn_ref[:, :H]
    w2 = wn_ref[:, H:]

    nb = neib_ref[...].reshape(M, E).astype(bf16)
    w2mat = w2mat_ref[...].astype(bf16)
    w1b = jnp.broadcast_to(w1.astype(bf16), (8, H))
    b2c = b2_ref[...].reshape(H, 1)

    NCHUNK = 4
    CM = M // NCHUNK
    rows_per_chunk = CM // N
    t1_parts = []
    for c in range(NCHUNK):
        nb_c = nb[c * CM:(c + 1) * CM, :]
        neiT = jax.lax.dot_general(w2mat, nb_c, (((1,), (1,)), ((), ())),
                                   preferred_element_type=jnp.float32)
        neiT = neiT + b2c
        t1c = jax.lax.dot_general(w1b, neiT.astype(bf16),
                                  (((1,), (0,)), ((), ())),
                                  preferred_element_type=jnp.float32)
        t1row = t1c[0:1, :]
        t1_parts.extend(t1row[:, b * N:(b + 1) * N]
                        for b in range(rows_per_chunk))
    t1 = jnp.concatenate(t1_parts, axis=0)
    w2b = jnp.broadcast_to(w2.astype(bf16), (8, H))
    t2 = jax.lax.dot_general(sg_ref[...].astype(bf16), w2b,
                             (((1,), (1,)), ((), ())),
                             preferred_element_type=jnp.float32)[:, 0:1]

    score = jax.nn.sigmoid(t1 + t2 + bn_ref[0, 0]) + pad_ref[...]
    score_scr[pl.ds(parity, 1)] = score.reshape(1, BBLK, N)


def _select_phase(nei_num_ref, sub_num_ref, id_ref, prob_ref, score_scr, parity):
    score_full = score_scr[pl.ds(parity, 1)].reshape(BBLK, N)
    idx_sorted, topsum = _select_half(score_full, BBLK)

    id_ref[...] = idx_sorted + sub_num_ref[...]
    nei_num_f = nei_num_ref[...].astype(jnp.float32)
    prob_ref[...] = topsum * nei_num_f * (1.0 / K)


def _select_half(score, rows):
    m = jnp.max(score, axis=1, keepdims=True)
    ex = jnp.exp(score - m)
    z = jnp.sum(ex, axis=1, keepdims=True)
    att = ex / z

    iota = jax.lax.broadcasted_iota(jnp.int32, (rows, N), 1)
    work = att
    topsum = jnp.zeros((rows, 1), dtype=jnp.float32)
    cols = []
    for _ in range(K):
        mx = jnp.max(work, axis=1, keepdims=True)
        eq = work == mx
        idxk = jnp.min(jnp.where(eq, iota, N), axis=1, keepdims=True)
        cols.append(idxk)
        topsum = topsum + mx
        work = jnp.where(iota == idxk, -1.0, work)
    idxmat = jnp.concatenate(cols, axis=1)

    ii = idxmat[:, :, None]
    jj = idxmat[:, None, :]
    rank = jnp.sum((jj < ii).astype(jnp.int32), axis=2)
    ko = jax.lax.broadcasted_iota(jnp.int32, (rows, K, K), 2)
    onehot = rank[:, :, None] == ko
    idx_sorted = jnp.sum(jnp.where(onehot, ii, 0), axis=1)
    return idx_sorted, topsum


def _body(neib_ref, sg_ref, pad_ref, nei_num_ref, sub_num_ref,
          w2mat_ref, b2_ref, wn_ref, bn_ref,
          id_ref, prob_ref, score_scr):
    i = pl.program_id(0)
    parity = jax.lax.rem(i, 2)

    _select_phase(nei_num_ref, sub_num_ref, id_ref, prob_ref, score_scr,
                  1 - parity)
    _score_phase(neib_ref, sg_ref, pad_ref, wn_ref, w2mat_ref, b2_ref,
                 bn_ref, score_scr, parity)


def _clamp_hi(i):
    return jnp.minimum(i, NSTEPS - 1)


def _lag(i):
    return jnp.maximum(i - 1, 0)


@jax.jit
def kernel(x, sub_graph, neibour, mask, nei_pad_mask, nei_num, sub_num, W2, b2, Wn, bn):
    del x, mask
    grid = (NSTEPS + 1,)
    out_id, out_prob = pl.pallas_call(
        _body,
        grid=grid,
        in_specs=[
            pl.BlockSpec((BBLK, N, E), lambda i: (_clamp_hi(i), 0, 0)),
            pl.BlockSpec((BBLK, H), lambda i: (_clamp_hi(i), 0)),
            pl.BlockSpec((BBLK, N), lambda i: (_clamp_hi(i), 0)),
            pl.BlockSpec((BBLK, 1), lambda i: (_lag(i), 0)),
            pl.BlockSpec((BBLK, 1), lambda i: (_lag(i), 0)),
            pl.BlockSpec((H, E), lambda i: (0, 0)),
            pl.BlockSpec((1, H), lambda i: (0, 0)),
            pl.BlockSpec((1, 2 * H), lambda i: (0, 0)),
            pl.BlockSpec((1, 1), lambda i: (0, 0)),
        ],
        out_specs=[
            pl.BlockSpec((BBLK, K), lambda i: (_lag(i), 0)),
            pl.BlockSpec((BBLK, 1), lambda i: (_lag(i), 0)),
        ],
        out_shape=[
            jax.ShapeDtypeStruct((B, K), jnp.int32),
            jax.ShapeDtypeStruct((B, 1), jnp.float32),
        ],
        scratch_shapes=[pltpu.VMEM((2, BBLK, N), jnp.float32)],
    )(neibour, sub_graph, nei_pad_mask, nei_num, sub_num,
      W2, b2.reshape(1, H), Wn, bn.reshape(1, 1))
    return out_id, out_prob.reshape(B)


# speedup vs baseline: 3.5956x; 1.0105x over previous
import jax
import jax.numpy as jnp
from jax.experimental import pallas as pl
from jax.experimental.pallas import tpu as pltpu

B, N, E, H, K = 1024, 200, 128, 256, 20
BBLK = 128
NSTEPS = B // BBLK


def _score_phase(neib_ref, sg_ref, pad_ref, wn_ref, w2mat_ref, b2_ref, bn_ref,
                 score_scr, parity):
    M = BBLK * N
    bf16 = jnp.bfloat16
    w1 = wn_ref[:, :H]
    w2 = wn_ref[:, H:]

    nb = neib_ref[...].reshape(M, E).astype(bf16)
    w2mat = w2mat_ref[...].astype(bf16)
    w1b = jnp.broadcast_to(w1.astype(bf16), (8, H))
    b2c = b2_ref[...].reshape(H, 1)

    NCHUNK = 4
    CM = M // NCHUNK
    rows_per_chunk = CM // N
    t1_parts = []
    for c in range(NCHUNK):
        nb_c = nb[c * CM:(c + 1) * CM, :]
        neiT = jax.lax.dot_general(w2mat, nb_c, (((1,), (1,)), ((), ())),
                                   preferred_element_type=jnp.float32)
        neiT = neiT + b2c
        t1c = jax.lax.dot_general(w1b, neiT.astype(bf16),
                                  (((1,), (0,)), ((), ())),
                                  preferred_element_type=jnp.float32)
        t1row = t1c[0:1, :]
        t1_parts.extend(t1row[:, b * N:(b + 1) * N]
                        for b in range(rows_per_chunk))
    t1 = jnp.concatenate(t1_parts, axis=0)
    w2b = jnp.broadcast_to(w2.astype(bf16), (8, H))
    t2 = jax.lax.dot_general(sg_ref[...].astype(bf16), w2b,
                             (((1,), (1,)), ((), ())),
                             preferred_element_type=jnp.float32)[:, 0:1]

    score = jax.nn.sigmoid(t1 + t2 + bn_ref[0, 0]) + pad_ref[...]
    score_scr[pl.ds(parity, 1)] = score.reshape(1, BBLK, N)


def _select_phase(nei_num_ref, sub_num_ref, id_ref, prob_ref, score_scr, parity):
    score_full = score_scr[pl.ds(parity, 1)].reshape(BBLK, N)
    idx_sorted, topsum = _select_block(score_full, BBLK)

    id_ref[...] = idx_sorted + sub_num_ref[...]
    nei_num_f = nei_num_ref[...].astype(jnp.float32)
    prob_ref[...] = topsum * nei_num_f * (1.0 / K)


def _select_block(score, rows):
    m = jnp.max(score, axis=1, keepdims=True)
    ex = jnp.exp(score - m)
    z = jnp.sum(ex, axis=1, keepdims=True)
    att = ex / z

    iota = jax.lax.broadcasted_iota(jnp.int32, (rows, N), 1)
    work = att
    topsum = jnp.zeros((rows, 1), dtype=jnp.float32)
    cols = []
    for _ in range(K):
        mx = jnp.max(work, axis=1, keepdims=True)
        eq = work == mx
        idxk = jnp.min(jnp.where(eq, iota, N), axis=1, keepdims=True)
        cols.append(idxk)
        topsum = topsum + mx
        work = jnp.where(iota == idxk, -1.0, work)
    idxmat = jnp.concatenate(cols, axis=1)

    ii = idxmat[:, :, None]
    jj = idxmat[:, None, :]
    rank = jnp.sum((jj < ii).astype(jnp.int32), axis=2)
    ko = jax.lax.broadcasted_iota(jnp.int32, (rows, K, K), 2)
    onehot = rank[:, :, None] == ko
    idx_sorted = jnp.sum(jnp.where(onehot, ii, 0), axis=1)
    return idx_sorted, topsum


def _body(neib_ref, sg_ref, pad_ref, nei_num_ref, sub_num_ref,
          w2mat_ref, b2_ref, wn_ref, bn_ref,
          id_ref, prob_ref, score_scr):
    i = pl.program_id(0)
    parity = jax.lax.rem(i, 2)

    _select_phase(nei_num_ref, sub_num_ref, id_ref, prob_ref, score_scr,
                  1 - parity)
    _score_phase(neib_ref, sg_ref, pad_ref, wn_ref, w2mat_ref, b2_ref,
                 bn_ref, score_scr, parity)


def _clamp_hi(i):
    return jnp.minimum(i, NSTEPS - 1)


def _lag(i):
    return jnp.maximum(i - 1, 0)


@jax.jit
def kernel(x, sub_graph, neibour, mask, nei_pad_mask, nei_num, sub_num, W2, b2, Wn, bn):
    del x, mask
    grid = (NSTEPS + 1,)
    out_id, out_prob = pl.pallas_call(
        _body,
        grid=grid,
        in_specs=[
            pl.BlockSpec((BBLK, N, E), lambda i: (_clamp_hi(i), 0, 0)),
            pl.BlockSpec((BBLK, H), lambda i: (_clamp_hi(i), 0)),
            pl.BlockSpec((BBLK, N), lambda i: (_clamp_hi(i), 0)),
            pl.BlockSpec((BBLK, 1), lambda i: (_lag(i), 0)),
            pl.BlockSpec((BBLK, 1), lambda i: (_lag(i), 0)),
            pl.BlockSpec((H, E), lambda i: (0, 0)),
            pl.BlockSpec((1, H), lambda i: (0, 0)),
            pl.BlockSpec((1, 2 * H), lambda i: (0, 0)),
            pl.BlockSpec((1, 1), lambda i: (0, 0)),
        ],
        out_specs=[
            pl.BlockSpec((BBLK, K), lambda i: (_lag(i), 0)),
            pl.BlockSpec((BBLK, 1), lambda i: (_lag(i), 0)),
        ],
        out_shape=[
            jax.ShapeDtypeStruct((B, K), jnp.int32),
            jax.ShapeDtypeStruct((B, 1), jnp.float32),
        ],
        scratch_shapes=[pltpu.VMEM((2, BBLK, N), jnp.float32)],
    )(neibour, sub_graph, nei_pad_mask, nei_num, sub_num,
      W2, b2.reshape(1, H), Wn, bn.reshape(1, 1))
    return out_id, out_prob.reshape(B)
